# baseline (speedup 1.0000x reference)
import functools

import jax
import jax.numpy as jnp
from jax import lax
from jax.experimental import pallas as pl
from jax.experimental.pallas import tpu as pltpu
from jax.experimental.pallas import tpu_sc as plsc

B = 16384
C = 10
L = 16
NC = 2
NS = 16
NW = NC * NS
B_PER_W = B // NW
GROUPS = B_PER_W // L
E_PER_W = B_PER_W * C
IDX_CHUNK = 128
N_CHUNKS = E_PER_W // IDX_CHUNK


def _sc_exp_sums(T_flat, bayes, partial_flat):
    mesh = plsc.VectorSubcoreMesh(core_axis_name="c", subcore_axis_name="s")

    @functools.partial(
        pl.kernel,
        mesh=mesh,
        out_type=jax.ShapeDtypeStruct((B,), jnp.float32),
        scratch_types=[
            pltpu.VMEM((B_PER_W,), jnp.int32),
            pltpu.VMEM((N_CHUNKS, IDX_CHUNK), jnp.int32),
            pltpu.VMEM((N_CHUNKS, IDX_CHUNK), jnp.int32),
            pltpu.VMEM((E_PER_W,), jnp.float32),
            pltpu.VMEM((E_PER_W,), jnp.int32),
            pltpu.VMEM((B_PER_W,), jnp.float32),
            pltpu.SemaphoreType.DMA,
        ],
    )
    def body(t_hbm, bayes_hbm, part_hbm, out_hbm, bayes_v, idx_v, idxp_v,
             q_v, p_v, out_v, sem):
        wid = lax.axis_index("s") * NC + lax.axis_index("c")
        base = pl.multiple_of(wid * B_PER_W, B_PER_W)

        pltpu.sync_copy(bayes_hbm.at[pl.ds(base, B_PER_W)], bayes_v)

        iota = lax.iota(jnp.int32, L)

        def idx_body(g, carry):
            goff = pl.multiple_of(g * L, L)
            bv = bayes_v[pl.ds(goff, L)]
            samp = base + goff + iota
            t_row0 = samp * (C * C) + bv * C
            p_row0 = samp * C
            flat = pl.multiple_of(g * (L * C), L)
            for c in range(C):
                pos = flat + c * L
                idx_v[pos // IDX_CHUNK, pl.ds(pos % IDX_CHUNK, L)] = t_row0 + c
                idxp_v[pos // IDX_CHUNK, pl.ds(pos % IDX_CHUNK, L)] = p_row0 + c
            return carry

        lax.fori_loop(0, GROUPS, idx_body, 0, unroll=2)

        gathers = [
            pltpu.async_copy(
                t_hbm.at[idx_v.at[j]],
                q_v.at[pl.ds(j * IDX_CHUNK, IDX_CHUNK)],
                sem,
            )
            for j in range(N_CHUNKS)
        ] + [
            pltpu.async_copy(
                part_hbm.at[idxp_v.at[j]],
                p_v.at[pl.ds(j * IDX_CHUNK, IDX_CHUNK)],
                sem,
            )
            for j in range(N_CHUNKS)
        ]
        for cp in gathers:
            cp.wait()

        def group_body(g, carry):
            goff = pl.multiple_of(g * L, L)
            flat = pl.multiple_of(g * (L * C), L)
            acc_neg = jnp.zeros((L,), jnp.float32)
            acc_pos = jnp.zeros((L,), jnp.float32)
            for c in range(C):
                q_c = q_v[pl.ds(flat + c * L, L)]
                p_c = p_v[pl.ds(flat + c * L, L)]
                e_pos = jnp.exp(q_c)
                e_neg = jnp.exp(-q_c)
                is_neg = p_c == 0
                acc_neg = acc_neg + jnp.where(is_neg, e_pos, 0.0)
                acc_pos = acc_pos + jnp.where(is_neg, 0.0, e_neg)
            out_v[pl.ds(goff, L)] = acc_neg * acc_pos
            return carry

        lax.fori_loop(0, GROUPS, group_body, 0)
        pltpu.sync_copy(out_v, out_hbm.at[pl.ds(base, B_PER_W)])

    return body(T_flat, bayes, partial_flat)


def _tc_mean_log1p(s):
    def tc_body(x_ref, o_ref):
        total = jnp.sum(jnp.log1p(x_ref[...]), keepdims=True)
        o_ref[...] = total * (1.0 / B)

    out = pl.pallas_call(
        tc_body,
        out_shape=jax.ShapeDtypeStruct((1, 1), jnp.float32),
    )(s.reshape(128, 128))
    return out[0, 0]


@jax.jit
def kernel(T, bayes, partial):
    exp_sum = _sc_exp_sums(
        T.reshape(B * C * C), bayes, partial.reshape(B * C))
    return _tc_mean_log1p(exp_sum)


# speedup vs baseline: 16.3412x; 16.3412x over previous
import functools

import jax
import jax.numpy as jnp
from jax.experimental import pallas as pl

B = 16384
C = 10
BB = 2048
GRID = B // BB


def _body(tp_ref, pp_ref, bayes_ref, o_ref):
    step = pl.program_id(0)

    t = tp_ref[...]
    p = pp_ref[...]
    bayes = bayes_ref[...]

    q = jnp.zeros((C, BB), jnp.float32)
    for r in range(C):
        q = jnp.where((bayes == r)[None, :], t[r], q)

    is_neg = p == 0
    acc_neg = jnp.sum(jnp.where(is_neg, jnp.exp(q), 0.0), axis=0)
    acc_pos = jnp.sum(jnp.where(is_neg, 0.0, jnp.exp(-q)), axis=0)
    part = jnp.sum(jnp.log1p(acc_neg * acc_pos), keepdims=True) * (1.0 / B)

    @pl.when(step == 0)
    def _():
        o_ref[...] = jnp.zeros_like(o_ref)

    o_ref[...] += part.reshape(1, 1)


@jax.jit
def kernel(T, bayes, partial):
    tp = jnp.transpose(T, (1, 2, 0))
    pp = partial.T
    out = pl.pallas_call(
        _body,
        grid=(GRID,),
        in_specs=[
            pl.BlockSpec((C, C, BB), lambda i: (0, 0, i)),
            pl.BlockSpec((C, BB), lambda i: (0, i)),
            pl.BlockSpec((BB,), lambda i: (i,)),
        ],
        out_specs=pl.BlockSpec((1, 1), lambda i: (0, 0)),
        out_shape=jax.ShapeDtypeStruct((1, 1), jnp.float32),
    )(tp, pp, bayes)
    return out[0, 0]


# speedup vs baseline: 17.2125x; 1.0533x over previous
import functools

import jax
import jax.numpy as jnp
from jax.experimental import pallas as pl

B = 16384
C = 10
BB = 2048
GRID = B // BB


def _body(tp_ref, pp_ref, bayes_ref, o_ref):
    step = pl.program_id(0)

    t = tp_ref[...]
    p = pp_ref[...]
    bayes = bayes_ref[...]

    bayes_b = jnp.broadcast_to(bayes[None, :], (C, BB))
    q = jnp.zeros((C, BB), jnp.float32)
    for r in range(C):
        q = jnp.where(bayes_b == r, t[r], q)

    is_neg = p == 0
    e = jnp.exp(jnp.where(is_neg, q, -q))
    acc_neg = jnp.sum(jnp.where(is_neg, e, 0.0), axis=0)
    acc_pos = jnp.sum(e, axis=0) - acc_neg
    part = jnp.sum(jnp.log1p(acc_neg * acc_pos), keepdims=True) * (1.0 / B)

    @pl.when(step == 0)
    def _():
        o_ref[...] = jnp.zeros_like(o_ref)

    o_ref[...] += part.reshape(1, 1)


@jax.jit
def kernel(T, bayes, partial):
    tp = jnp.transpose(T, (1, 2, 0))
    pp = partial.T
    out = pl.pallas_call(
        _body,
        grid=(GRID,),
        in_specs=[
            pl.BlockSpec((C, C, BB), lambda i: (0, 0, i)),
            pl.BlockSpec((C, BB), lambda i: (0, i)),
            pl.BlockSpec((BB,), lambda i: (i,)),
        ],
        out_specs=pl.BlockSpec((1, 1), lambda i: (0, 0)),
        out_shape=jax.ShapeDtypeStruct((1, 1), jnp.float32),
    )(tp, pp, bayes)
    return out[0, 0]


# speedup vs baseline: 21.6328x; 1.2568x over previous
import functools

import jax
import jax.numpy as jnp
from jax.experimental import pallas as pl

B = 16384
C = 10
BB = 4096
GRID = B // BB


def _body(tp_ref, pp_ref, bayes_ref, o_ref):
    step = pl.program_id(0)

    t = tp_ref[...]
    p = pp_ref[...]
    bayes = bayes_ref[...]

    bayes_b = jnp.broadcast_to(bayes[None, :], (C, BB))
    q = jnp.zeros((C, BB), jnp.float32)
    for r in range(C):
        q = jnp.where(bayes_b == r, t[r], q)

    is_neg = p == 0
    e = jnp.exp(jnp.where(is_neg, q, -q))
    acc_neg = jnp.sum(jnp.where(is_neg, e, 0.0), axis=0)
    acc_pos = jnp.sum(e, axis=0) - acc_neg
    part = jnp.sum(jnp.log1p(acc_neg * acc_pos), keepdims=True) * (1.0 / B)

    @pl.when(step == 0)
    def _():
        o_ref[...] = jnp.zeros_like(o_ref)

    o_ref[...] += part.reshape(1, 1)


@jax.jit
def kernel(T, bayes, partial):
    tp = jnp.transpose(T, (1, 2, 0))
    pp = partial.T
    out = pl.pallas_call(
        _body,
        grid=(GRID,),
        in_specs=[
            pl.BlockSpec((C, C, BB), lambda i: (0, 0, i)),
            pl.BlockSpec((C, BB), lambda i: (0, i)),
            pl.BlockSpec((BB,), lambda i: (i,)),
        ],
        out_specs=pl.BlockSpec((1, 1), lambda i: (0, 0)),
        out_shape=jax.ShapeDtypeStruct((1, 1), jnp.float32),
    )(tp, pp, bayes)
    return out[0, 0]


# speedup vs baseline: 22.6064x; 1.0450x over previous
import functools

import jax
import jax.numpy as jnp
from jax.experimental import pallas as pl

B = 16384
C = 10
BB = 8192
GRID = B // BB


def _body(tp_ref, pp_ref, bayes_ref, o_ref):
    step = pl.program_id(0)

    t = tp_ref[...]
    p = pp_ref[...]
    bayes = bayes_ref[...]

    bayes_b = jnp.broadcast_to(bayes[None, :], (C, BB))
    q = jnp.zeros((C, BB), jnp.float32)
    for r in range(C):
        q = jnp.where(bayes_b == r, t[r], q)

    is_neg = p == 0
    e = jnp.exp(jnp.where(is_neg, q, -q))
    acc_neg = jnp.sum(jnp.where(is_neg, e, 0.0), axis=0)
    acc_pos = jnp.sum(e, axis=0) - acc_neg
    part = jnp.sum(jnp.log1p(acc_neg * acc_pos), keepdims=True) * (1.0 / B)

    @pl.when(step == 0)
    def _():
        o_ref[...] = jnp.zeros_like(o_ref)

    o_ref[...] += part.reshape(1, 1)


@jax.jit
def kernel(T, bayes, partial):
    tp = jnp.transpose(T, (1, 2, 0))
    pp = partial.T
    out = pl.pallas_call(
        _body,
        grid=(GRID,),
        in_specs=[
            pl.BlockSpec((C, C, BB), lambda i: (0, 0, i)),
            pl.BlockSpec((C, BB), lambda i: (0, i)),
            pl.BlockSpec((BB,), lambda i: (i,)),
        ],
        out_specs=pl.BlockSpec((1, 1), lambda i: (0, 0)),
        out_shape=jax.ShapeDtypeStruct((1, 1), jnp.float32),
    )(tp, pp, bayes)
    return out[0, 0]
